# Initial kernel scaffold; baseline (speedup 1.0000x reference)
import functools

import jax
import jax.numpy as jnp
from jax import lax
from jax.experimental import pallas as pl
from jax.experimental.pallas import tpu as pltpu
from jax.experimental.pallas import tpu_sc as plsc

N_NODES = 10000
N_EDGES = 320000
D = 128

NC = 2
NS = 16
NW = NC * NS
CH = 128
NCHUNK = -(-N_EDGES // (NW * CH))
EPT = NCHUNK * CH
EPAD = NW * EPT

NPAD = 10016
RPT = NPAD // NS
NDEG = 10240
DPT = NDEG // NS

_mesh = plsc.VectorSubcoreMesh(core_axis_name="c", subcore_axis_name="s")


@functools.partial(
    pl.kernel,
    out_type=jax.ShapeDtypeStruct((NC, NDEG), jnp.float32),
    mesh=_mesh,
    scratch_types=[
        pltpu.VMEM_SHARED((NDEG,), jnp.float32),
        pltpu.VMEM((NCHUNK, CH), jnp.int32),
        pltpu.VMEM((CH,), jnp.float32),
    ],
)
def _deg_kernel(dst_hbm, ones_hbm, zeros_hbm, degp_hbm, deg_sh, dst_v, ones_v):
    cid = lax.axis_index("c")
    sid = lax.axis_index("s")
    wid = cid * NS + sid

    pltpu.sync_copy(zeros_hbm.at[pl.ds(sid * DPT, DPT)],
                    deg_sh.at[pl.ds(sid * DPT, DPT)])
    pltpu.sync_copy(ones_hbm, ones_v)
    pltpu.sync_copy(dst_hbm.at[wid], dst_v)
    plsc.subcore_barrier()

    def body(j, _):
        pltpu.sync_copy(ones_v, deg_sh.at[dst_v.at[j]], add=True)
        return 0

    lax.fori_loop(0, NCHUNK, body, 0)
    plsc.subcore_barrier()

    pltpu.sync_copy(deg_sh.at[pl.ds(sid * DPT, DPT)],
                    degp_hbm.at[cid, pl.ds(sid * DPT, DPT)])


@functools.partial(
    pl.kernel,
    out_type=jax.ShapeDtypeStruct((NC, NPAD, D), jnp.float32),
    mesh=_mesh,
    scratch_types=[
        pltpu.VMEM_SHARED((NPAD, D), jnp.float32),
        pltpu.VMEM((NCHUNK, CH), jnp.int32),
        pltpu.VMEM((NCHUNK, CH), jnp.int32),
        pltpu.VMEM((2, CH, D), jnp.float32),
        pltpu.SemaphoreType.DMA,
        pltpu.SemaphoreType.DMA,
    ],
)
def _scatter_kernel(hs_hbm, src_hbm, dst_hbm, zrows_hbm, p_hbm,
                    agg_sh, src_v, dst_v, rows_v, sem0, sem1):
    cid = lax.axis_index("c")
    sid = lax.axis_index("s")
    wid = cid * NS + sid

    pltpu.sync_copy(zrows_hbm.at[pl.ds(sid * RPT, RPT)],
                    agg_sh.at[pl.ds(sid * RPT, RPT)])
    pltpu.sync_copy(src_hbm.at[wid], src_v)
    pltpu.sync_copy(dst_hbm.at[wid], dst_v)
    plsc.subcore_barrier()

    pltpu.async_copy(hs_hbm.at[src_v.at[0]], rows_v.at[0], sem0)

    def body(j, _):
        buf = lax.rem(j, 2)

        @pl.when(j + 1 < NCHUNK)
        def _():
            @pl.when(buf == 0)
            def _():
                pltpu.async_copy(hs_hbm.at[src_v.at[j + 1]], rows_v.at[1], sem1)

            @pl.when(buf == 1)
            def _():
                pltpu.async_copy(hs_hbm.at[src_v.at[j + 1]], rows_v.at[0], sem0)

        @pl.when(buf == 0)
        def _():
            pltpu.make_async_copy(hs_hbm.at[src_v.at[j]], rows_v.at[0], sem0).wait()
            pltpu.sync_copy(rows_v.at[0], agg_sh.at[dst_v.at[j]], add=True)

        @pl.when(buf == 1)
        def _():
            pltpu.make_async_copy(hs_hbm.at[src_v.at[j]], rows_v.at[1], sem1).wait()
            pltpu.sync_copy(rows_v.at[1], agg_sh.at[dst_v.at[j]], add=True)

        return 0

    lax.fori_loop(0, NCHUNK, body, 0)
    plsc.subcore_barrier()

    pltpu.sync_copy(agg_sh.at[pl.ds(sid * RPT, RPT)],
                    p_hbm.at[cid, pl.ds(sid * RPT, RPT)])


def _pre_body(data_ref, wpre_ref, wpost_ref, degp_ref, hs_ref, isd_ref):
    wc = jnp.dot(wpre_ref[...], wpost_ref[...], preferred_element_type=jnp.float32)
    h = jnp.dot(data_ref[...], wc, preferred_element_type=jnp.float32)
    deg = degp_ref[0] + degp_ref[1]
    isd = lax.rsqrt(jnp.maximum(deg, 1.0))
    isd_ref[...] = isd[:NPAD]
    hs_ref[0:N_NODES, :] = h * isd[:N_NODES]
    hs_ref[N_NODES:NPAD, :] = jnp.zeros((NPAD - N_NODES, D), jnp.float32)


_pre_call = pl.pallas_call(
    _pre_body,
    out_shape=(
        jax.ShapeDtypeStruct((NPAD, D), jnp.float32),
        jax.ShapeDtypeStruct((NPAD, 1), jnp.float32),
    ),
)


def _post_body(p_ref, isd_ref, out_ref):
    s = p_ref[0] + p_ref[1]
    out_ref[...] = s[:N_NODES] * isd_ref[0:N_NODES, :]


_post_call = pl.pallas_call(
    _post_body,
    out_shape=jax.ShapeDtypeStruct((N_NODES, D), jnp.float32),
)


@jax.jit
def kernel(data, edge_index, W_pre, W_post):
    src = edge_index[0]
    dst = edge_index[1]
    pad = EPAD - N_EDGES
    fill = jnp.full((pad,), N_NODES, jnp.int32)
    src_p = jnp.concatenate([src, fill]).reshape(NW, NCHUNK, CH)
    dst_p = jnp.concatenate([dst, fill]).reshape(NW, NCHUNK, CH)

    degp = _deg_kernel(dst_p, jnp.ones((CH,), jnp.float32),
                       jnp.zeros((NDEG,), jnp.float32))
    degp_col = degp[:, :, None]
    hs, isd = _pre_call(data, W_pre, W_post, degp_col)
    p = _scatter_kernel(hs, src_p, dst_p,
                        jnp.zeros((NPAD, D), jnp.float32))
    return _post_call(p, isd)


# speedup vs baseline: 18.5739x; 18.5739x over previous
import functools

import jax
import jax.numpy as jnp
from jax import lax
from jax.experimental import pallas as pl
from jax.experimental.pallas import tpu as pltpu
from jax.experimental.pallas import tpu_sc as plsc

N_NODES = 10000
N_EDGES = 320000
D = 128

NC = 2
NS = 16
NW = NC * NS
CH = 128
NCHUNK = -(-N_EDGES // (NW * CH))
EPT = NCHUNK * CH
EPAD = NW * EPT

NPAD = 10112
RPT = NPAD // NS
NDEG = 10240
DPT = NDEG // NS

_mesh = plsc.VectorSubcoreMesh(core_axis_name="c", subcore_axis_name="s")


@functools.partial(
    pl.kernel,
    out_type=jax.ShapeDtypeStruct((NC, NDEG), jnp.float32),
    mesh=_mesh,
    scratch_types=[
        pltpu.VMEM_SHARED((NDEG,), jnp.float32),
        pltpu.VMEM((NCHUNK, CH), jnp.int32),
        pltpu.VMEM((CH,), jnp.float32),
    ],
)
def _deg_kernel(dst_hbm, ones_hbm, zeros_hbm, degp_hbm, deg_sh, dst_v, ones_v):
    cid = lax.axis_index("c")
    sid = lax.axis_index("s")
    wid = cid * NS + sid

    pltpu.sync_copy(zeros_hbm.at[pl.ds(sid * DPT, DPT)],
                    deg_sh.at[pl.ds(sid * DPT, DPT)])
    pltpu.sync_copy(ones_hbm, ones_v)
    pltpu.sync_copy(dst_hbm.at[wid], dst_v)
    plsc.subcore_barrier()

    def body(j, _):
        pltpu.sync_copy(ones_v, deg_sh.at[dst_v.at[j]], add=True)
        return 0

    lax.fori_loop(0, NCHUNK, body, 0)
    plsc.subcore_barrier()

    pltpu.sync_copy(deg_sh.at[pl.ds(sid * DPT, DPT)],
                    degp_hbm.at[cid, pl.ds(sid * DPT, DPT)])


@functools.partial(
    pl.kernel,
    out_type=jax.ShapeDtypeStruct((NC, NPAD, D), jnp.float32),
    mesh=_mesh,
    scratch_types=[
        pltpu.VMEM_SHARED((NPAD, D), jnp.float32),
        pltpu.VMEM((NCHUNK, CH), jnp.int32),
        pltpu.VMEM((2, CH), jnp.int32),
        pltpu.VMEM((2, CH, D), jnp.float32),
        pltpu.SemaphoreType.DMA,
        pltpu.SemaphoreType.DMA,
        pltpu.SemaphoreType.DMA,
        pltpu.SemaphoreType.DMA,
    ],
)
def _scatter_kernel(hs_hbm, src_hbm, dst_hbm, zrows_hbm, p_hbm,
                    agg_sh, dst_v, sidx_v, rows_v, gsem0, gsem1, isem0, isem1):
    cid = lax.axis_index("c")
    sid = lax.axis_index("s")
    wid = cid * NS + sid

    pltpu.sync_copy(zrows_hbm.at[pl.ds(sid * RPT, RPT)],
                    agg_sh.at[pl.ds(sid * RPT, RPT)])
    pltpu.sync_copy(dst_hbm.at[wid], dst_v)
    plsc.subcore_barrier()

    pltpu.sync_copy(src_hbm.at[wid, 0], sidx_v.at[0])
    pltpu.async_copy(hs_hbm.at[sidx_v.at[0]], rows_v.at[0], gsem0)
    pltpu.async_copy(src_hbm.at[wid, 1], sidx_v.at[1], isem1)

    def body(j, _):
        buf = lax.rem(j, 2)

        def halfstep(b, nb, gsem_b, gsem_nb, isem_b, isem_nb):
            @pl.when(j + 1 < NCHUNK)
            def _():
                pltpu.make_async_copy(src_hbm.at[wid, j + 1], sidx_v.at[nb],
                                      isem_nb).wait()
                pltpu.async_copy(hs_hbm.at[sidx_v.at[nb]], rows_v.at[nb], gsem_nb)

            pltpu.make_async_copy(hs_hbm.at[sidx_v.at[b]], rows_v.at[b],
                                  gsem_b).wait()

            @pl.when(j + 2 < NCHUNK)
            def _():
                pltpu.async_copy(src_hbm.at[wid, j + 2], sidx_v.at[b], isem_b)

            pltpu.sync_copy(rows_v.at[b], agg_sh.at[dst_v.at[j]], add=True)

        @pl.when(buf == 0)
        def _():
            halfstep(0, 1, gsem0, gsem1, isem0, isem1)

        @pl.when(buf == 1)
        def _():
            halfstep(1, 0, gsem1, gsem0, isem1, isem0)

        return 0

    lax.fori_loop(0, NCHUNK, body, 0)
    plsc.subcore_barrier()

    pltpu.sync_copy(agg_sh.at[pl.ds(sid * RPT, RPT)],
                    p_hbm.at[cid, pl.ds(sid * RPT, RPT)])


def _pre_body(data_ref, wpre_ref, wpost_ref, degp_ref, hs_ref, isd_ref):
    wc = jnp.dot(wpre_ref[...], wpost_ref[...], preferred_element_type=jnp.float32)
    h = jnp.dot(data_ref[...], wc, preferred_element_type=jnp.float32)
    deg = degp_ref[0] + degp_ref[1]
    isd = lax.rsqrt(jnp.maximum(deg, 1.0))
    isd_ref[...] = isd[:NPAD]
    hs_ref[0:N_NODES, :] = h * isd[:N_NODES]
    hs_ref[N_NODES:NPAD, :] = jnp.zeros((NPAD - N_NODES, D), jnp.float32)


_pre_call = pl.pallas_call(
    _pre_body,
    out_shape=(
        jax.ShapeDtypeStruct((NPAD, D), jnp.float32),
        jax.ShapeDtypeStruct((NPAD, 1), jnp.float32),
    ),
)


def _post_body(p_ref, isd_ref, out_ref):
    s = p_ref[0] + p_ref[1]
    out_ref[...] = s[:N_NODES] * isd_ref[0:N_NODES, :]


_post_call = pl.pallas_call(
    _post_body,
    out_shape=jax.ShapeDtypeStruct((N_NODES, D), jnp.float32),
)


@jax.jit
def kernel(data, edge_index, W_pre, W_post):
    src = edge_index[0]
    dst = edge_index[1]
    pad = EPAD - N_EDGES
    fill = jnp.full((pad,), N_NODES, jnp.int32)
    src_p = jnp.concatenate([src, fill]).reshape(NW, NCHUNK, CH)
    dst_p = jnp.concatenate([dst, fill]).reshape(NW, NCHUNK, CH)

    degp = _deg_kernel(dst_p, jnp.ones((CH,), jnp.float32),
                       jnp.zeros((NDEG,), jnp.float32))
    degp_col = degp[:, :, None]
    hs, isd = _pre_call(data, W_pre, W_post, degp_col)
    p = _scatter_kernel(hs, src_p, dst_p,
                        jnp.zeros((NPAD, D), jnp.float32))
    return _post_call(p, isd)


# speedup vs baseline: 25.6092x; 1.3788x over previous
import functools

import jax
import jax.numpy as jnp
from jax import lax
from jax.experimental import pallas as pl
from jax.experimental.pallas import tpu as pltpu
from jax.experimental.pallas import tpu_sc as plsc

N_NODES = 10000
N_EDGES = 320000
D = 128

NC = 2
NS = 16
NW = NC * NS
CH = 128
TOTCH = -(-N_EDGES // (NS * CH)) * NS
EPAD = TOTCH * CH

F0 = 44
F1 = TOTCH // NS - F0
FMAX = max(F0, F1)

NPAD = 10112
RPT = NPAD // NS
NDEG = 10240
DPT = NDEG // NS

_mesh = plsc.VectorSubcoreMesh(core_axis_name="c", subcore_axis_name="s")


@functools.partial(
    pl.kernel,
    out_type=jax.ShapeDtypeStruct((NC, NDEG), jnp.float32),
    mesh=_mesh,
    scratch_types=[
        pltpu.VMEM_SHARED((NDEG,), jnp.float32),
        pltpu.VMEM((FMAX, CH), jnp.int32),
        pltpu.VMEM((CH,), jnp.float32),
    ],
)
def _deg_kernel(dst0_hbm, dst1_hbm, ones_hbm, zeros_hbm, degp_hbm,
                deg_sh, dst_v, ones_v):
    cid = lax.axis_index("c")
    sid = lax.axis_index("s")

    pltpu.sync_copy(zeros_hbm.at[pl.ds(sid * DPT, DPT)],
                    deg_sh.at[pl.ds(sid * DPT, DPT)])
    pltpu.sync_copy(ones_hbm, ones_v)

    @pl.when(cid == 0)
    def _():
        pltpu.sync_copy(dst0_hbm.at[sid], dst_v.at[pl.ds(0, F0)])

    @pl.when(cid == 1)
    def _():
        pltpu.sync_copy(dst1_hbm.at[sid], dst_v.at[pl.ds(0, F1)])

    plsc.subcore_barrier()
    nch = jnp.where(cid == 0, F0, F1)

    def body(j, _):
        pltpu.sync_copy(ones_v, deg_sh.at[dst_v.at[j]], add=True)
        return 0

    lax.fori_loop(0, nch, body, 0)
    plsc.subcore_barrier()

    pltpu.sync_copy(deg_sh.at[pl.ds(sid * DPT, DPT)],
                    degp_hbm.at[cid, pl.ds(sid * DPT, DPT)])


@functools.partial(
    pl.kernel,
    out_type=jax.ShapeDtypeStruct((NC, NPAD, D), jnp.float32),
    mesh=_mesh,
    scratch_types=[
        pltpu.VMEM_SHARED((NPAD, D), jnp.float32),
        pltpu.VMEM((FMAX, CH), jnp.int32),
        pltpu.VMEM((2, CH), jnp.int32),
        pltpu.VMEM((2, CH, D), jnp.float32),
        pltpu.SemaphoreType.DMA,
        pltpu.SemaphoreType.DMA,
        pltpu.SemaphoreType.DMA,
        pltpu.SemaphoreType.DMA,
    ],
)
def _scatter_kernel(hs_hbm, src0_hbm, src1_hbm, dst0_hbm, dst1_hbm, p_hbm,
                    agg_sh, dst_v, sidx_v, rows_v, gsem0, gsem1, isem0, isem1):
    cid = lax.axis_index("c")
    sid = lax.axis_index("s")

    z16 = jnp.zeros((16,), jnp.float32)

    def zbody(r, _):
        for c in range(D // 16):
            rows_v[0, r, pl.ds(c * 16, 16)] = z16
        return 0

    lax.fori_loop(0, CH, zbody, 0)
    for k in range(4):
        pltpu.sync_copy(rows_v.at[0],
                        agg_sh.at[pl.ds(sid * RPT + k * CH, CH)])
    pltpu.sync_copy(rows_v.at[0, pl.ds(0, RPT - 4 * CH)],
                    agg_sh.at[pl.ds(sid * RPT + 4 * CH, RPT - 4 * CH)])

    @pl.when(cid == 0)
    def _():
        pltpu.sync_copy(dst0_hbm.at[sid], dst_v.at[pl.ds(0, F0)])

    @pl.when(cid == 1)
    def _():
        pltpu.sync_copy(dst1_hbm.at[sid], dst_v.at[pl.ds(0, F1)])

    plsc.subcore_barrier()

    def run(src_hbm, nch):
        pltpu.sync_copy(src_hbm.at[sid, 0], sidx_v.at[0])
        pltpu.async_copy(hs_hbm.at[sidx_v.at[0]], rows_v.at[0], gsem0)
        pltpu.async_copy(src_hbm.at[sid, 1], sidx_v.at[1], isem1)

        def body(j, _):
            buf = lax.rem(j, 2)

            def halfstep(b, nb, gsem_b, gsem_nb, isem_b, isem_nb):
                @pl.when(j + 1 < nch)
                def _():
                    pltpu.make_async_copy(src_hbm.at[sid, j + 1], sidx_v.at[nb],
                                          isem_nb).wait()
                    pltpu.async_copy(hs_hbm.at[sidx_v.at[nb]], rows_v.at[nb],
                                     gsem_nb)

                pltpu.make_async_copy(hs_hbm.at[sidx_v.at[b]], rows_v.at[b],
                                      gsem_b).wait()

                @pl.when(j + 2 < nch)
                def _():
                    pltpu.async_copy(src_hbm.at[sid, j + 2], sidx_v.at[b],
                                     isem_b)

                pltpu.sync_copy(rows_v.at[b], agg_sh.at[dst_v.at[j]], add=True)

            @pl.when(buf == 0)
            def _():
                halfstep(0, 1, gsem0, gsem1, isem0, isem1)

            @pl.when(buf == 1)
            def _():
                halfstep(1, 0, gsem1, gsem0, isem1, isem0)

            return 0

        lax.fori_loop(0, nch, body, 0)

    @pl.when(cid == 0)
    def _():
        run(src0_hbm, F0)

    @pl.when(cid == 1)
    def _():
        run(src1_hbm, F1)

    plsc.subcore_barrier()

    pltpu.sync_copy(agg_sh.at[pl.ds(sid * RPT, RPT)],
                    p_hbm.at[cid, pl.ds(sid * RPT, RPT)])


def _pre_body(data_ref, wpre_ref, wpost_ref, degp_ref, hs_ref, isd_ref):
    wc = jnp.dot(wpre_ref[...], wpost_ref[...], preferred_element_type=jnp.float32)
    h = jnp.dot(data_ref[...], wc, preferred_element_type=jnp.float32)
    deg = degp_ref[0] + degp_ref[1]
    isd = lax.rsqrt(jnp.maximum(deg, 1.0))
    isd_ref[...] = isd[:NPAD]
    hs_ref[0:N_NODES, :] = h * isd[:N_NODES]
    hs_ref[N_NODES:NPAD, :] = jnp.zeros((NPAD - N_NODES, D), jnp.float32)


_pre_call = pl.pallas_call(
    _pre_body,
    out_shape=(
        jax.ShapeDtypeStruct((NPAD, D), jnp.float32),
        jax.ShapeDtypeStruct((NPAD, 1), jnp.float32),
    ),
)


def _post_body(p_ref, isd_ref, out_ref):
    s = p_ref[0] + p_ref[1]
    out_ref[...] = s[:N_NODES] * isd_ref[0:N_NODES, :]


_post_call = pl.pallas_call(
    _post_body,
    out_shape=jax.ShapeDtypeStruct((N_NODES, D), jnp.float32),
)


@jax.jit
def kernel(data, edge_index, W_pre, W_post):
    src = edge_index[0]
    dst = edge_index[1]
    pad = EPAD - N_EDGES
    fill = jnp.full((pad,), N_NODES, jnp.int32)
    src_f = jnp.concatenate([src, fill])
    dst_f = jnp.concatenate([dst, fill])
    n0 = NS * F0 * CH
    src0 = src_f[:n0].reshape(NS, F0, CH)
    src1 = src_f[n0:].reshape(NS, F1, CH)
    dst0 = dst_f[:n0].reshape(NS, F0, CH)
    dst1 = dst_f[n0:].reshape(NS, F1, CH)

    degp = _deg_kernel(dst0, dst1, jnp.ones((CH,), jnp.float32),
                       jnp.zeros((NDEG,), jnp.float32))
    degp_col = degp[:, :, None]
    hs, isd = _pre_call(data, W_pre, W_post, degp_col)
    p = _scatter_kernel(hs, src0, src1, dst0, dst1)
    return _post_call(p, isd)
